# Initial kernel scaffold; baseline (speedup 1.0000x reference)
import jax
import jax.numpy as jnp
from jax.experimental import pallas as pl


def kernel(x, adj, W_gc1, b_gc1, W_gc2, b_gc2, W_l1, b_l1, W_l2, b_l2, W_l3, b_l3):
    raise NotImplementedError("write your pallas kernel here")


# speedup vs baseline: 1.0106x; 1.0106x over previous
import jax
import jax.numpy as jnp
from jax.experimental import pallas as pl


def _prep_kernel(x_ref, w1_ref, out_ref):
    out_ref[...] = jnp.dot(x_ref[...], w1_ref[...],
                           preferred_element_type=jnp.float32)


def _pass1_kernel(adj_ref, s1_ref, b1_ref, w2_ref, out_ref):
    h = jnp.dot(adj_ref[...], s1_ref[...], preferred_element_type=jnp.float32)
    h = jnp.maximum(h + b1_ref[...], 0.0)
    out_ref[...] = jnp.dot(h, w2_ref[...], preferred_element_type=jnp.float32)


def _pass2_kernel(adj_ref, s2_ref, b2_ref, wl1_ref, bl1_ref, wl2_ref,
                  bl2_ref, wl3_ref, bl3_ref, out_ref):
    z = jnp.dot(adj_ref[...], s2_ref[...], preferred_element_type=jnp.float32)
    z = jnp.maximum(z + b2_ref[...], 0.0)
    h = jnp.dot(z, wl1_ref[...], preferred_element_type=jnp.float32)
    h = jnp.maximum(h + bl1_ref[...], 0.0)
    h = jnp.dot(h, wl2_ref[...], preferred_element_type=jnp.float32)
    h = jnp.maximum(h + bl2_ref[...], 0.0)
    o = jnp.dot(h, wl3_ref[...], preferred_element_type=jnp.float32)
    o = o + bl3_ref[...]
    m = jnp.max(o, axis=1, keepdims=True)
    lse = jnp.log(jnp.sum(jnp.exp(o - m), axis=1, keepdims=True))
    out_ref[...] = o - m - lse


def _row_block(n):
    for r in (512, 400, 256, 200, 128, 80, 64, 40, 16, 8):
        if n % r == 0 and r % 8 == 0:
            return r
    return n


def kernel(x, adj, W_gc1, b_gc1, W_gc2, b_gc2, W_l1, b_l1, W_l2, b_l2,
           W_l3, b_l3):
    n = adj.shape[-1]
    hid = W_gc1.shape[1]
    classes = W_l3.shape[1]
    x2 = x.reshape(n, x.shape[-1])
    adj2 = adj.reshape(n, n)
    b1 = b_gc1.reshape(1, hid)
    b2 = b_gc2.reshape(1, hid)
    bl1 = b_l1.reshape(1, -1)
    bl2 = b_l2.reshape(1, -1)
    bl3 = b_l3.reshape(1, -1)

    r = _row_block(n)
    nb = n // r

    s1 = pl.pallas_call(
        _prep_kernel,
        out_shape=jax.ShapeDtypeStruct((n, hid), jnp.float32),
    )(x2, W_gc1)

    full = lambda shape: pl.BlockSpec(shape, lambda i: (0, 0))
    rows = lambda w: pl.BlockSpec((r, w), lambda i: (i, 0))

    s2 = pl.pallas_call(
        _pass1_kernel,
        grid=(nb,),
        in_specs=[rows(n), full((n, hid)), full((1, hid)), full((hid, hid))],
        out_specs=rows(hid),
        out_shape=jax.ShapeDtypeStruct((n, hid), jnp.float32),
    )(adj2, s1, b1, W_gc2)

    out = pl.pallas_call(
        _pass2_kernel,
        grid=(nb,),
        in_specs=[rows(n), full((n, hid)), full((1, hid)),
                  full(W_l1.shape), full((1, W_l1.shape[1])),
                  full(W_l2.shape), full((1, W_l2.shape[1])),
                  full(W_l3.shape), full((1, classes))],
        out_specs=rows(classes),
        out_shape=jax.ShapeDtypeStruct((n, classes), jnp.float32),
    )(adj2, s2, b2, W_l1, bl1, W_l2, bl2, W_l3, bl3)

    return jnp.transpose(out[None], (0, 2, 1))
